# baseline (speedup 1.0000x reference)
import functools

import jax
import jax.numpy as jnp
from jax.experimental import pallas as pl
from jax.experimental.pallas import tpu as pltpu

BC = 2048
CP = 100096
GR = 64


def _argmax_body(x_ref, am_ref, m_ref, *, n_cols):
    j = pl.program_id(0)
    x = x_ref[...]
    cols = j * BC + jax.lax.broadcasted_iota(jnp.int32, x.shape, 1)
    x = jnp.where(cols < n_cols, x, -jnp.inf)
    bm = jnp.max(x, axis=1, keepdims=True)
    bi = jnp.min(jnp.where(x == bm, cols, jnp.int32(2**31 - 1)),
                 axis=1, keepdims=True)

    @pl.when(j == 0)
    def _():
        m_ref[...] = bm
        am_ref[...] = bi

    @pl.when(j > 0)
    def _():
        prev = m_ref[...]
        upd = bm > prev
        m_ref[...] = jnp.where(upd, bm, prev)
        am_ref[...] = jnp.where(upd, bi, am_ref[...])


def _onehot_body(am_ref, o_ref):
    cols = jax.lax.broadcasted_iota(jnp.int32, o_ref.shape, 1)
    o_ref[...] = (cols == am_ref[...]).astype(jnp.float32)


@jax.jit
def kernel(Xsoft):
    rows, n_cols = Xsoft.shape
    nb = pl.cdiv(n_cols, BC)

    am = pl.pallas_call(
        functools.partial(_argmax_body, n_cols=n_cols),
        grid=(nb,),
        in_specs=[pl.BlockSpec((rows, BC), lambda j: (0, j))],
        out_specs=pl.BlockSpec((rows, 1), lambda j: (0, 0)),
        out_shape=jax.ShapeDtypeStruct((rows, 1), jnp.int32),
        scratch_shapes=[pltpu.VMEM((rows, 1), jnp.float32)],
        compiler_params=pltpu.CompilerParams(
            dimension_semantics=("arbitrary",)),
    )(Xsoft)

    padded = pl.pallas_call(
        _onehot_body,
        grid=(rows // GR,),
        in_specs=[pl.BlockSpec((GR, 1), lambda i: (i, 0))],
        out_specs=pl.BlockSpec((GR, CP), lambda i: (i, 0)),
        out_shape=jax.ShapeDtypeStruct((rows, CP), jnp.float32),
        compiler_params=pltpu.CompilerParams(
            dimension_semantics=("arbitrary",)),
    )(am)
    return jax.lax.slice(padded, (0, 0), (rows, n_cols))


# speedup vs baseline: 1.0106x; 1.0106x over previous
import functools

import jax
import jax.numpy as jnp
from jax.experimental import pallas as pl
from jax.experimental.pallas import tpu as pltpu

BC = 4096
CP = 100096
GR = 64


def _argmax_body(x_ref, am_ref, m_ref, *, n_cols):
    j = pl.program_id(0)
    x = x_ref[...]
    cols = j * BC + jax.lax.broadcasted_iota(jnp.int32, x.shape, 1)
    x = jnp.where(cols < n_cols, x, -jnp.inf)
    bm = jnp.max(x, axis=1, keepdims=True)
    bi = jnp.min(jnp.where(x == bm, cols, jnp.int32(2**31 - 1)),
                 axis=1, keepdims=True)

    @pl.when(j == 0)
    def _():
        m_ref[...] = bm
        am_ref[...] = bi

    @pl.when(j > 0)
    def _():
        prev = m_ref[...]
        upd = bm > prev
        m_ref[...] = jnp.where(upd, bm, prev)
        am_ref[...] = jnp.where(upd, bi, am_ref[...])


def _onehot_body(am_ref, o_ref):
    cols = jax.lax.broadcasted_iota(jnp.int32, o_ref.shape, 1)
    o_ref[...] = (cols == am_ref[...]).astype(jnp.float32)


@jax.jit
def kernel(Xsoft):
    rows, n_cols = Xsoft.shape
    nb = pl.cdiv(n_cols, BC)

    am = pl.pallas_call(
        functools.partial(_argmax_body, n_cols=n_cols),
        grid=(nb,),
        in_specs=[pl.BlockSpec((rows, BC), lambda j: (0, j))],
        out_specs=pl.BlockSpec((rows, 1), lambda j: (0, 0)),
        out_shape=jax.ShapeDtypeStruct((rows, 1), jnp.int32),
        scratch_shapes=[pltpu.VMEM((rows, 1), jnp.float32)],
        compiler_params=pltpu.CompilerParams(
            dimension_semantics=("arbitrary",)),
    )(Xsoft)

    padded = pl.pallas_call(
        _onehot_body,
        grid=(rows // GR,),
        in_specs=[pl.BlockSpec((GR, 1), lambda i: (i, 0))],
        out_specs=pl.BlockSpec((GR, CP), lambda i: (i, 0)),
        out_shape=jax.ShapeDtypeStruct((rows, CP), jnp.float32),
        compiler_params=pltpu.CompilerParams(
            dimension_semantics=("arbitrary",)),
    )(am)
    return jax.lax.slice(padded, (0, 0), (rows, n_cols))
